# Initial kernel scaffold; baseline (speedup 1.0000x reference)
import jax
import jax.numpy as jnp
from jax.experimental import pallas as pl


def kernel(x, edge_index, birth_table, gender_table, symp_tables, W1, b1, W2, b2, Wl, bl):
    raise NotImplementedError("write your pallas kernel here")


# speedup vs baseline: 35.7000x; 35.7000x over previous
import functools

import jax
import jax.numpy as jnp
from jax import lax
from jax.experimental import pallas as pl
from jax.experimental.pallas import tpu as pltpu
from jax.experimental.pallas import tpu_sc as plsc

N = 100000
E = 3200000
HID = 16

NC = 2
NS = 16
NW = NC * NS

BB = 128
CH = 16
NSC = 49
EP = BB * CH * NSC * NW
RW = CH * NSC

NP = 100096
RPW = NP // NS
ZB = 368

_f32 = jnp.float32


def _sc_mesh():
    return plsc.VectorSubcoreMesh(
        core_axis_name="c", subcore_axis_name="s", num_cores=NC, num_subcores=NS
    )


_SC_PARAMS = pltpu.CompilerParams(use_tc_tiling_on_sc=False)


def _zero_acc(zbuf, acc, s):
    def zrow(i, carry):
        zbuf[i, :] = jnp.zeros((16,), _f32)
        return carry

    lax.fori_loop(0, ZB, zrow, 0)
    r0 = s * RPW
    for r in range(RPW // ZB):
        pltpu.sync_copy(zbuf, acc.at[pl.ds(r0 + r * ZB, ZB)])


def _copy_out(acc, out0, out1, c, s):
    r0 = s * RPW

    @pl.when(c == 0)
    def _():
        pltpu.sync_copy(acc.at[pl.ds(r0, RPW)], out0.at[pl.ds(r0, RPW)])

    @pl.when(c == 1)
    def _():
        pltpu.sync_copy(acc.at[pl.ds(r0, RPW)], out1.at[pl.ds(r0, RPW)])


def _deg_body(dst_hbm, out0, out1, acc, didx, erows, zbuf):
    c = lax.axis_index("c")
    s = lax.axis_index("s")
    w = c * NS + s
    _zero_acc(zbuf, acc, s)

    def er(i, carry):
        lane = lax.broadcasted_iota(jnp.int32, (16,), 0)
        erows[i, :] = jnp.where(lane == 0, 1.0, 0.0).astype(_f32)
        return carry

    lax.fori_loop(0, BB, er, 0)
    plsc.subcore_barrier()

    row0 = w * RW

    def superchunk(sc, carry):
        pltpu.sync_copy(dst_hbm.at[pl.ds(row0 + sc * CH, CH)], didx)

        def bat(j, carry2):
            pltpu.sync_copy(erows, acc.at[didx.at[j]], add=True)
            return carry2

        lax.fori_loop(0, CH, bat, 0)
        return carry

    lax.fori_loop(0, NSC, superchunk, 0)
    plsc.subcore_barrier()
    _copy_out(acc, out0, out1, c, s)


def _layer_body(g_hbm, src_hbm, dst_hbm, out0, out1, acc, sidx, didx, rows, zbuf):
    c = lax.axis_index("c")
    s = lax.axis_index("s")
    w = c * NS + s
    _zero_acc(zbuf, acc, s)
    plsc.subcore_barrier()

    row0 = w * RW

    def superchunk(sc, carry):
        base = row0 + sc * CH
        pltpu.sync_copy(src_hbm.at[pl.ds(base, CH)], sidx)
        pltpu.sync_copy(dst_hbm.at[pl.ds(base, CH)], didx)

        def bat(j, carry2):
            pltpu.sync_copy(g_hbm.at[sidx.at[j]], rows)
            pltpu.sync_copy(rows, acc.at[didx.at[j]], add=True)
            return carry2

        lax.fori_loop(0, CH, bat, 0)
        return carry

    lax.fori_loop(0, NSC, superchunk, 0)
    plsc.subcore_barrier()
    _copy_out(acc, out0, out1, c, s)


@jax.jit
def _sc_deg(dst2d):
    fn = pl.kernel(
        _deg_body,
        out_type=(
            jax.ShapeDtypeStruct((NP, 16), _f32),
            jax.ShapeDtypeStruct((NP, 16), _f32),
        ),
        mesh=_sc_mesh(),
        scratch_types=[
            pltpu.VMEM_SHARED((NP, 16), _f32),
            pltpu.VMEM((CH, BB), jnp.int32),
            pltpu.VMEM((BB, 16), _f32),
            pltpu.VMEM((ZB, 16), _f32),
        ],
        compiler_params=_SC_PARAMS,
    )
    return fn(dst2d)


@jax.jit
def _sc_layer(g, src2d, dst2d):
    fn = pl.kernel(
        _layer_body,
        out_type=(
            jax.ShapeDtypeStruct((NP, 16), _f32),
            jax.ShapeDtypeStruct((NP, 16), _f32),
        ),
        mesh=_sc_mesh(),
        scratch_types=[
            pltpu.VMEM_SHARED((NP, 16), _f32),
            pltpu.VMEM((CH, BB), jnp.int32),
            pltpu.VMEM((CH, BB), jnp.int32),
            pltpu.VMEM((BB, 16), _f32),
            pltpu.VMEM((ZB, 16), _f32),
        ],
        compiler_params=_SC_PARAMS,
    )
    return fn(g, src2d, dst2d)


BN = 1000


def _tc_emb_body(x_ref, d0_ref, d1_ref, c1_ref, p1_ref, q1_ref, g1_ref, dinv_ref):
    xb = x_ref[...]
    deg = 1.0 + d0_ref[:, 0:1] + d1_ref[:, 0:1]
    dinv = lax.rsqrt(deg)
    hw = (
        c1_ref[...]
        + jnp.dot(xb, p1_ref[...], preferred_element_type=_f32)
        + jnp.dot(xb * xb, q1_ref[...], preferred_element_type=_f32)
    )
    g1_ref[...] = hw * dinv
    dinv_ref[...] = dinv


def _tc_mid_body(g1_ref, a0_ref, a1_ref, dinv_ref, w2_ref, b1_ref, g2_ref):
    dinv = dinv_ref[...]
    pre = b1_ref[...] + dinv * (g1_ref[...] + a0_ref[...] + a1_ref[...])
    h1 = jnp.where(pre > 0.0, pre, jnp.exp(pre) - 1.0)
    g2_ref[...] = jnp.dot(h1, w2_ref[...], preferred_element_type=_f32) * dinv


def _tc_head_body(g2_ref, a0_ref, a1_ref, dinv_ref, wl_ref, b2_ref, bl_ref, y_ref):
    h2 = b2_ref[...] + dinv_ref[...] * (g2_ref[...] + a0_ref[...] + a1_ref[...])
    y_ref[...] = jnp.dot(h2, wl_ref[...], preferred_element_type=_f32) + bl_ref[...]


def _row_spec(cols):
    return pl.BlockSpec((BN, cols), lambda i: (i, 0))


def _full_spec(shape):
    return pl.BlockSpec(shape, lambda i: tuple(0 for _ in shape))


def _tc_emb(x, d0, d1, c1, p1, q1):
    return pl.pallas_call(
        _tc_emb_body,
        grid=(N // BN,),
        in_specs=[
            _row_spec(17),
            _row_spec(16),
            _row_spec(16),
            _full_spec((1, 16)),
            _full_spec((17, 16)),
            _full_spec((17, 16)),
        ],
        out_specs=[_row_spec(16), _row_spec(1)],
        out_shape=[
            jax.ShapeDtypeStruct((N, 16), _f32),
            jax.ShapeDtypeStruct((N, 1), _f32),
        ],
    )(x, d0, d1, c1, p1, q1)


def _tc_mid(g1, a0, a1, dinv, w2, b1r):
    return pl.pallas_call(
        _tc_mid_body,
        grid=(N // BN,),
        in_specs=[
            _row_spec(16),
            _row_spec(16),
            _row_spec(16),
            _row_spec(1),
            _full_spec((16, 16)),
            _full_spec((1, 16)),
        ],
        out_specs=_row_spec(16),
        out_shape=jax.ShapeDtypeStruct((N, 16), _f32),
    )(g1, a0, a1, dinv, w2, b1r)


def _tc_head(g2, a0, a1, dinv, wl, b2r, blr):
    return pl.pallas_call(
        _tc_head_body,
        grid=(N // BN,),
        in_specs=[
            _row_spec(16),
            _row_spec(16),
            _row_spec(16),
            _row_spec(1),
            _full_spec((16, 1)),
            _full_spec((1, 16)),
            _full_spec((1, 1)),
        ],
        out_specs=_row_spec(1),
        out_shape=jax.ShapeDtypeStruct((N, 1), _f32),
    )(g2, a0, a1, dinv, wl, b2r, blr)


def kernel(x, edge_index, birth_table, gender_table, symp_tables, W1, b1, W2, b2, Wl, bl):
    T0 = symp_tables[:, 0, :]
    T1 = symp_tables[:, 1, :]
    T2 = symp_tables[:, 2, :]
    Bc = -1.5 * T0 + 2.0 * T1 - 0.5 * T2
    Cc = 0.5 * T0 - 1.0 * T1 + 0.5 * T2
    P = jnp.zeros((17, 32), _f32)
    P = P.at[1, :].set(birth_table[1] - birth_table[0])
    P = P.at[4, :].add(gender_table[1] - gender_table[0])
    P = P.at[2:17, :].add(Bc / 15.0)
    Q = jnp.zeros((17, 32), _f32).at[2:17, :].set(Cc / 15.0)
    C0 = birth_table[0] + gender_table[0] + jnp.sum(T0, axis=0) / 15.0
    c1 = (C0 @ W1 / 3.0).reshape(1, HID)
    P1 = P @ W1 / 3.0
    Q1 = Q @ W1 / 3.0

    pad = EP - E
    src = jnp.concatenate([edge_index[0], jnp.zeros((pad,), jnp.int32)])
    dst = jnp.concatenate([edge_index[1], jnp.full((pad,), N, jnp.int32)])
    src2d = src.reshape(EP // BB, BB)
    dst2d = dst.reshape(EP // BB, BB)

    d0, d1 = _sc_deg(dst2d)
    g1, dinv = _tc_emb(x, d0, d1, c1, P1, Q1)
    a0, a1 = _sc_layer(g1, src2d, dst2d)
    g2 = _tc_mid(g1, a0, a1, dinv, W2, b1.reshape(1, HID))
    c0p, c1p = _sc_layer(g2, src2d, dst2d)
    return _tc_head(g2, c0p, c1p, dinv, Wl, b2.reshape(1, HID), bl.reshape(1, 1))


# speedup vs baseline: 66.0630x; 1.8505x over previous
import functools

import jax
import jax.numpy as jnp
from jax import lax
from jax.experimental import pallas as pl
from jax.experimental.pallas import tpu as pltpu
from jax.experimental.pallas import tpu_sc as plsc

N = 100000
E = 3200000
HID = 16

NC = 2
NS = 16
NW = NC * NS

BB = 128
CH = 16
NSC = 49
EP = BB * CH * NSC * NW
RW = CH * NSC
NBUF = 8

NP = 100096
RPW = NP // NS
ZB = 368

_f32 = jnp.float32


def _sc_mesh():
    return plsc.VectorSubcoreMesh(
        core_axis_name="c", subcore_axis_name="s", num_cores=NC, num_subcores=NS
    )


_SC_PARAMS = pltpu.CompilerParams(use_tc_tiling_on_sc=False)


def _zero_acc(zbuf, acc, s):
    def zrow(i, carry):
        zbuf[i, :] = jnp.zeros((16,), _f32)
        return carry

    lax.fori_loop(0, ZB, zrow, 0)
    r0 = s * RPW
    for r in range(RPW // ZB):
        pltpu.sync_copy(zbuf, acc.at[pl.ds(r0 + r * ZB, ZB)])


def _copy_out(acc, out0, out1, c, s):
    r0 = s * RPW

    @pl.when(c == 0)
    def _():
        pltpu.sync_copy(acc.at[pl.ds(r0, RPW)], out0.at[pl.ds(r0, RPW)])

    @pl.when(c == 1)
    def _():
        pltpu.sync_copy(acc.at[pl.ds(r0, RPW)], out1.at[pl.ds(r0, RPW)])


def _deg_body(dst_hbm, out0, out1, acc, didx, erows, zbuf, isem):
    c = lax.axis_index("c")
    s = lax.axis_index("s")
    w = c * NS + s
    _zero_acc(zbuf, acc, s)

    def er(i, carry):
        lane = lax.broadcasted_iota(jnp.int32, (16,), 0)
        erows[i, :] = jnp.where(lane == 0, 1.0, 0.0).astype(_f32)
        return carry

    lax.fori_loop(0, BB, er, 0)
    plsc.subcore_barrier()

    row0 = w * RW
    pltpu.sync_copy(dst_hbm.at[pl.ds(row0, CH)], didx.at[0])
    for sc_i in range(NSC):
        slot = sc_i % 2
        if sc_i + 1 < NSC:
            pltpu.async_copy(
                dst_hbm.at[pl.ds(row0 + (sc_i + 1) * CH, CH)],
                didx.at[1 - slot], isem,
            )

        def bat(j, carry, slot=slot):
            pltpu.sync_copy(erows, acc.at[didx.at[slot, j]], add=True)
            return carry

        lax.fori_loop(0, CH, bat, 0)
        if sc_i + 1 < NSC:
            pltpu.make_async_copy(
                dst_hbm.at[pl.ds(row0, CH)], didx.at[1 - slot], isem
            ).wait()
    plsc.subcore_barrier()
    _copy_out(acc, out0, out1, c, s)


def _layer_body(g_hbm, src_hbm, dst_hbm, out0, out1, acc, sidx, didx, rows, zbuf, gsem, isem):
    c = lax.axis_index("c")
    s = lax.axis_index("s")
    w = c * NS + s
    _zero_acc(zbuf, acc, s)
    plsc.subcore_barrier()

    row0 = w * RW

    def _issue(slot, j, r):
        pltpu.async_copy(g_hbm.at[sidx.at[slot, j]], rows.at[r], gsem.at[r])

    def _wait_gather(r):
        pltpu.make_async_copy(
            g_hbm.at[sidx.at[0, 0]], rows.at[r], gsem.at[r]
        ).wait()

    pltpu.sync_copy(src_hbm.at[pl.ds(row0, CH)], sidx.at[0])
    pltpu.sync_copy(dst_hbm.at[pl.ds(row0, CH)], didx.at[0])
    for sc_i in range(NSC):
        slot = sc_i % 2
        if sc_i + 1 < NSC:
            base = row0 + (sc_i + 1) * CH
            pltpu.async_copy(src_hbm.at[pl.ds(base, CH)], sidx.at[1 - slot], isem)
            pltpu.async_copy(dst_hbm.at[pl.ds(base, CH)], didx.at[1 - slot], isem)
        for p in range(NBUF):
            _issue(slot, p, p)

        def bat(j, carry, slot=slot):
            r = lax.rem(j, NBUF)
            _wait_gather(r)
            pltpu.sync_copy(rows.at[r], acc.at[didx.at[slot, j]], add=True)

            @pl.when(j + NBUF < CH)
            def _():
                _issue(slot, j + NBUF, r)

            return carry

        lax.fori_loop(0, CH, bat, 0)
        if sc_i + 1 < NSC:
            pltpu.make_async_copy(
                src_hbm.at[pl.ds(row0, CH)], sidx.at[1 - slot], isem
            ).wait()
            pltpu.make_async_copy(
                dst_hbm.at[pl.ds(row0, CH)], didx.at[1 - slot], isem
            ).wait()
    plsc.subcore_barrier()
    _copy_out(acc, out0, out1, c, s)


@jax.jit
def _sc_deg(dst2d):
    fn = pl.kernel(
        _deg_body,
        out_type=(
            jax.ShapeDtypeStruct((NP, 16), _f32),
            jax.ShapeDtypeStruct((NP, 16), _f32),
        ),
        mesh=_sc_mesh(),
        scratch_types=[
            pltpu.VMEM_SHARED((NP, 16), _f32),
            pltpu.VMEM((2, CH, BB), jnp.int32),
            pltpu.VMEM((BB, 16), _f32),
            pltpu.VMEM((ZB, 16), _f32),
            pltpu.SemaphoreType.DMA,
        ],
        compiler_params=_SC_PARAMS,
    )
    return fn(dst2d)


@jax.jit
def _sc_layer(g, src2d, dst2d):
    fn = pl.kernel(
        _layer_body,
        out_type=(
            jax.ShapeDtypeStruct((NP, 16), _f32),
            jax.ShapeDtypeStruct((NP, 16), _f32),
        ),
        mesh=_sc_mesh(),
        scratch_types=[
            pltpu.VMEM_SHARED((NP, 16), _f32),
            pltpu.VMEM((2, CH, BB), jnp.int32),
            pltpu.VMEM((2, CH, BB), jnp.int32),
            pltpu.VMEM((NBUF, BB, 16), _f32),
            pltpu.VMEM((ZB, 16), _f32),
            pltpu.SemaphoreType.DMA((NBUF,)),
            pltpu.SemaphoreType.DMA,
        ],
        compiler_params=_SC_PARAMS,
    )
    return fn(g, src2d, dst2d)


BN = 1000


def _tc_emb_body(x_ref, d0_ref, d1_ref, c1_ref, p1_ref, q1_ref, g1_ref, dinv_ref):
    xb = x_ref[...]
    deg = 1.0 + d0_ref[:, 0:1] + d1_ref[:, 0:1]
    dinv = lax.rsqrt(deg)
    hw = (
        c1_ref[...]
        + jnp.dot(xb, p1_ref[...], preferred_element_type=_f32)
        + jnp.dot(xb * xb, q1_ref[...], preferred_element_type=_f32)
    )
    g1_ref[...] = hw * dinv
    dinv_ref[...] = dinv


def _tc_mid_body(g1_ref, a0_ref, a1_ref, dinv_ref, w2_ref, b1_ref, g2_ref):
    dinv = dinv_ref[...]
    pre = b1_ref[...] + dinv * (g1_ref[...] + a0_ref[...] + a1_ref[...])
    h1 = jnp.where(pre > 0.0, pre, jnp.exp(pre) - 1.0)
    g2_ref[...] = jnp.dot(h1, w2_ref[...], preferred_element_type=_f32) * dinv


def _tc_head_body(g2_ref, a0_ref, a1_ref, dinv_ref, wl_ref, b2_ref, bl_ref, y_ref):
    h2 = b2_ref[...] + dinv_ref[...] * (g2_ref[...] + a0_ref[...] + a1_ref[...])
    y_ref[...] = jnp.dot(h2, wl_ref[...], preferred_element_type=_f32) + bl_ref[...]


def _row_spec(cols):
    return pl.BlockSpec((BN, cols), lambda i: (i, 0))


def _full_spec(shape):
    return pl.BlockSpec(shape, lambda i: tuple(0 for _ in shape))


def _tc_emb(x, d0, d1, c1, p1, q1):
    return pl.pallas_call(
        _tc_emb_body,
        grid=(N // BN,),
        in_specs=[
            _row_spec(17),
            _row_spec(16),
            _row_spec(16),
            _full_spec((1, 16)),
            _full_spec((17, 16)),
            _full_spec((17, 16)),
        ],
        out_specs=[_row_spec(16), _row_spec(1)],
        out_shape=[
            jax.ShapeDtypeStruct((N, 16), _f32),
            jax.ShapeDtypeStruct((N, 1), _f32),
        ],
    )(x, d0, d1, c1, p1, q1)


def _tc_mid(g1, a0, a1, dinv, w2, b1r):
    return pl.pallas_call(
        _tc_mid_body,
        grid=(N // BN,),
        in_specs=[
            _row_spec(16),
            _row_spec(16),
            _row_spec(16),
            _row_spec(1),
            _full_spec((16, 16)),
            _full_spec((1, 16)),
        ],
        out_specs=_row_spec(16),
        out_shape=jax.ShapeDtypeStruct((N, 16), _f32),
    )(g1, a0, a1, dinv, w2, b1r)


def _tc_head(g2, a0, a1, dinv, wl, b2r, blr):
    return pl.pallas_call(
        _tc_head_body,
        grid=(N // BN,),
        in_specs=[
            _row_spec(16),
            _row_spec(16),
            _row_spec(16),
            _row_spec(1),
            _full_spec((16, 1)),
            _full_spec((1, 16)),
            _full_spec((1, 1)),
        ],
        out_specs=_row_spec(1),
        out_shape=jax.ShapeDtypeStruct((N, 1), _f32),
    )(g2, a0, a1, dinv, wl, b2r, blr)


def kernel(x, edge_index, birth_table, gender_table, symp_tables, W1, b1, W2, b2, Wl, bl):
    T0 = symp_tables[:, 0, :]
    T1 = symp_tables[:, 1, :]
    T2 = symp_tables[:, 2, :]
    Bc = -1.5 * T0 + 2.0 * T1 - 0.5 * T2
    Cc = 0.5 * T0 - 1.0 * T1 + 0.5 * T2
    P = jnp.zeros((17, 32), _f32)
    P = P.at[1, :].set(birth_table[1] - birth_table[0])
    P = P.at[4, :].add(gender_table[1] - gender_table[0])
    P = P.at[2:17, :].add(Bc / 15.0)
    Q = jnp.zeros((17, 32), _f32).at[2:17, :].set(Cc / 15.0)
    C0 = birth_table[0] + gender_table[0] + jnp.sum(T0, axis=0) / 15.0
    c1 = (C0 @ W1 / 3.0).reshape(1, HID)
    P1 = P @ W1 / 3.0
    Q1 = Q @ W1 / 3.0

    pad = EP - E
    src = jnp.concatenate([edge_index[0], jnp.zeros((pad,), jnp.int32)])
    dst = jnp.concatenate([edge_index[1], jnp.full((pad,), N, jnp.int32)])
    src2d = src.reshape(EP // BB, BB)
    dst2d = dst.reshape(EP // BB, BB)

    d0, d1 = _sc_deg(dst2d)
    g1, dinv = _tc_emb(x, d0, d1, c1, P1, Q1)
    a0, a1 = _sc_layer(g1, src2d, dst2d)
    g2 = _tc_mid(g1, a0, a1, dinv, W2, b1.reshape(1, HID))
    c0p, c1p = _sc_layer(g2, src2d, dst2d)
    return _tc_head(g2, c0p, c1p, dinv, Wl, b2.reshape(1, HID), bl.reshape(1, 1))


# speedup vs baseline: 117.3706x; 1.7766x over previous
import functools

import jax
import jax.numpy as jnp
from jax import lax
from jax.experimental import pallas as pl
from jax.experimental.pallas import tpu as pltpu
from jax.experimental.pallas import tpu_sc as plsc

N = 100000
E = 3200000
HID = 16

NC = 2
NS = 16
NW = NC * NS

BB = 128
CH = 16
NSC = 49
EP = BB * CH * NSC * NW
RW = CH * NSC
NBUF = 8

NP = 100096
RPW = NP // NS
ZB = 368

_f32 = jnp.float32


def _sc_mesh():
    return plsc.VectorSubcoreMesh(
        core_axis_name="c", subcore_axis_name="s", num_cores=NC, num_subcores=NS
    )


_SC_PARAMS = pltpu.CompilerParams(use_tc_tiling_on_sc=False)


def _zero_acc(zbuf, acc, s):
    def zrow(i, carry):
        zbuf[i, :] = jnp.zeros((16,), _f32)
        return carry

    lax.fori_loop(0, ZB, zrow, 0)
    r0 = s * RPW
    for r in range(RPW // ZB):
        pltpu.sync_copy(zbuf, acc.at[pl.ds(r0 + r * ZB, ZB)])


def _copy_out(acc, out0, out1, c, s):
    r0 = s * RPW

    @pl.when(c == 0)
    def _():
        pltpu.sync_copy(acc.at[pl.ds(r0, RPW)], out0.at[pl.ds(r0, RPW)])

    @pl.when(c == 1)
    def _():
        pltpu.sync_copy(acc.at[pl.ds(r0, RPW)], out1.at[pl.ds(r0, RPW)])


def _deg_body(dst_hbm, out0, out1, acc, didx, erows, zbuf, isem):
    c = lax.axis_index("c")
    s = lax.axis_index("s")
    w = c * NS + s
    _zero_acc(zbuf, acc, s)

    def er(i, carry):
        erows[i, :] = jnp.full((16,), 1.0, _f32)
        return carry

    lax.fori_loop(0, BB, er, 0)
    plsc.subcore_barrier()

    row0 = w * RW
    pltpu.sync_copy(dst_hbm.at[pl.ds(row0, CH)], didx.at[0])
    for sc_i in range(NSC):
        slot = sc_i % 2
        if sc_i + 1 < NSC:
            pltpu.async_copy(
                dst_hbm.at[pl.ds(row0 + (sc_i + 1) * CH, CH)],
                didx.at[1 - slot], isem,
            )

        def bat(j, carry, slot=slot):
            pltpu.sync_copy(erows, acc.at[didx.at[slot, j]], add=True)
            return carry

        lax.fori_loop(0, CH, bat, 0)
        if sc_i + 1 < NSC:
            pltpu.make_async_copy(
                dst_hbm.at[pl.ds(row0, CH)], didx.at[1 - slot], isem
            ).wait()
    plsc.subcore_barrier()
    _copy_out(acc, out0, out1, c, s)


def _layer_body(g_hbm, src_hbm, dst_hbm, out0, out1, acc, sidx, didx, rows, zbuf, gsem, isem):
    c = lax.axis_index("c")
    s = lax.axis_index("s")
    w = c * NS + s
    _zero_acc(zbuf, acc, s)
    plsc.subcore_barrier()

    row0 = w * RW

    def _issue(slot, j, r):
        pltpu.async_copy(g_hbm.at[sidx.at[slot, j]], rows.at[r], gsem.at[r])

    def _wait_gather(r):
        pltpu.make_async_copy(
            g_hbm.at[sidx.at[0, 0]], rows.at[r], gsem.at[r]
        ).wait()

    pltpu.sync_copy(src_hbm.at[pl.ds(row0, CH)], sidx.at[0])
    pltpu.sync_copy(dst_hbm.at[pl.ds(row0, CH)], didx.at[0])
    for sc_i in range(NSC):
        slot = sc_i % 2
        if sc_i + 1 < NSC:
            base = row0 + (sc_i + 1) * CH
            pltpu.async_copy(src_hbm.at[pl.ds(base, CH)], sidx.at[1 - slot], isem)
            pltpu.async_copy(dst_hbm.at[pl.ds(base, CH)], didx.at[1 - slot], isem)
        for p in range(NBUF):
            _issue(slot, p, p)

        def bat(j, carry, slot=slot):
            r = lax.rem(j, NBUF)
            _wait_gather(r)
            pltpu.sync_copy(rows.at[r], acc.at[didx.at[slot, j]], add=True)

            @pl.when(j + NBUF < CH)
            def _():
                _issue(slot, j + NBUF, r)

            return carry

        lax.fori_loop(0, CH, bat, 0)
        if sc_i + 1 < NSC:
            pltpu.make_async_copy(
                src_hbm.at[pl.ds(row0, CH)], sidx.at[1 - slot], isem
            ).wait()
            pltpu.make_async_copy(
                dst_hbm.at[pl.ds(row0, CH)], didx.at[1 - slot], isem
            ).wait()
    plsc.subcore_barrier()
    _copy_out(acc, out0, out1, c, s)


@jax.jit
def _sc_deg(dst2d):
    fn = pl.kernel(
        _deg_body,
        out_type=(
            jax.ShapeDtypeStruct((NP, 16), _f32),
            jax.ShapeDtypeStruct((NP, 16), _f32),
        ),
        mesh=_sc_mesh(),
        scratch_types=[
            pltpu.VMEM_SHARED((NP, 16), _f32),
            pltpu.VMEM((2, CH, BB), jnp.int32),
            pltpu.VMEM((BB, 16), _f32),
            pltpu.VMEM((ZB, 16), _f32),
            pltpu.SemaphoreType.DMA,
        ],
        compiler_params=_SC_PARAMS,
    )
    return fn(dst2d)


@jax.jit
def _sc_layer(g, src2d, dst2d):
    fn = pl.kernel(
        _layer_body,
        out_type=(
            jax.ShapeDtypeStruct((NP, 16), _f32),
            jax.ShapeDtypeStruct((NP, 16), _f32),
        ),
        mesh=_sc_mesh(),
        scratch_types=[
            pltpu.VMEM_SHARED((NP, 16), _f32),
            pltpu.VMEM((2, CH, BB), jnp.int32),
            pltpu.VMEM((2, CH, BB), jnp.int32),
            pltpu.VMEM((NBUF, BB, 16), _f32),
            pltpu.VMEM((ZB, 16), _f32),
            pltpu.SemaphoreType.DMA((NBUF,)),
            pltpu.SemaphoreType.DMA,
        ],
        compiler_params=_SC_PARAMS,
    )
    return fn(g, src2d, dst2d)


NPK = NP * 16 // 128
BNP = 3128
XW = 136


def _tc_emb_body(x_ref, a0_ref, a1_ref, k_ref, kq_ref, c1_ref, g1_ref, dinv_ref):
    xb = x_ref[...]
    dinv = lax.rsqrt(1.0 + a0_ref[...] + a1_ref[...])
    hw = (
        c1_ref[...]
        + jnp.dot(xb, k_ref[...], preferred_element_type=_f32)
        + jnp.dot(xb * xb, kq_ref[...], preferred_element_type=_f32)
    )
    g1_ref[...] = hw * dinv
    dinv_ref[...] = dinv


def _tc_mid_body(g1_ref, a0_ref, a1_ref, dinv_ref, bd2_ref, b1_ref, g2_ref):
    dinv = dinv_ref[...]
    pre = b1_ref[...] + dinv * (g1_ref[...] + a0_ref[...] + a1_ref[...])
    h1 = jnp.where(pre > 0.0, pre, jnp.exp(pre) - 1.0)
    g2_ref[...] = jnp.dot(h1, bd2_ref[...], preferred_element_type=_f32) * dinv


def _tc_head_body(g2_ref, a0_ref, a1_ref, dinv_ref, s_ref, b2_ref, bl_ref, y_ref):
    h2 = b2_ref[...] + dinv_ref[...] * (g2_ref[...] + a0_ref[...] + a1_ref[...])
    y_ref[...] = jnp.dot(h2, s_ref[...], preferred_element_type=_f32) + bl_ref[...]


def _row_spec(cols):
    return pl.BlockSpec((BNP, cols), lambda i: (i, 0))


def _full_spec(shape):
    return pl.BlockSpec(shape, lambda i: tuple(0 for _ in shape))


def _tc_emb(xp, a0p, a1p, kmat, kqmat, c1p):
    return pl.pallas_call(
        _tc_emb_body,
        grid=(NPK // BNP,),
        in_specs=[
            _row_spec(XW),
            _row_spec(128),
            _row_spec(128),
            _full_spec((XW, 128)),
            _full_spec((XW, 128)),
            _full_spec((1, 128)),
        ],
        out_specs=[_row_spec(128), _row_spec(128)],
        out_shape=[
            jax.ShapeDtypeStruct((NPK, 128), _f32),
            jax.ShapeDtypeStruct((NPK, 128), _f32),
        ],
    )(xp, a0p, a1p, kmat, kqmat, c1p)


def _tc_mid(g1p, a0p, a1p, dinvp, bd2, b1p):
    return pl.pallas_call(
        _tc_mid_body,
        grid=(NPK // BNP,),
        in_specs=[
            _row_spec(128),
            _row_spec(128),
            _row_spec(128),
            _row_spec(128),
            _full_spec((128, 128)),
            _full_spec((1, 128)),
        ],
        out_specs=_row_spec(128),
        out_shape=jax.ShapeDtypeStruct((NPK, 128), _f32),
    )(g1p, a0p, a1p, dinvp, bd2, b1p)


def _tc_head(g2p, c0p, c1pp, dinvp, smat, b2p, blr):
    return pl.pallas_call(
        _tc_head_body,
        grid=(NPK // BNP,),
        in_specs=[
            _row_spec(128),
            _row_spec(128),
            _row_spec(128),
            _row_spec(128),
            _full_spec((128, 8)),
            _full_spec((1, 128)),
            _full_spec((1, 1)),
        ],
        out_specs=_row_spec(8),
        out_shape=jax.ShapeDtypeStruct((NPK, 8), _f32),
    )(g2p, c0p, c1pp, dinvp, smat, b2p, blr)


def kernel(x, edge_index, birth_table, gender_table, symp_tables, W1, b1, W2, b2, Wl, bl):
    T0 = symp_tables[:, 0, :]
    T1 = symp_tables[:, 1, :]
    T2 = symp_tables[:, 2, :]
    Bc = -1.5 * T0 + 2.0 * T1 - 0.5 * T2
    Cc = 0.5 * T0 - 1.0 * T1 + 0.5 * T2
    P = jnp.zeros((17, 32), _f32)
    P = P.at[1, :].set(birth_table[1] - birth_table[0])
    P = P.at[4, :].add(gender_table[1] - gender_table[0])
    P = P.at[2:17, :].add(Bc / 15.0)
    Q = jnp.zeros((17, 32), _f32).at[2:17, :].set(Cc / 15.0)
    C0 = birth_table[0] + gender_table[0] + jnp.sum(T0, axis=0) / 15.0
    c1 = C0 @ W1 / 3.0
    P1 = P @ W1 / 3.0
    Q1 = Q @ W1 / 3.0

    from jax.scipy.linalg import block_diag

    kmat = block_diag(*([P1] * 8))
    kqmat = block_diag(*([Q1] * 8))
    bd2 = block_diag(*([W2] * 8))
    smat = block_diag(*([Wl] * 8))
    c1p = jnp.tile(c1, 8).reshape(1, 128)
    b1p = jnp.tile(b1, 8).reshape(1, 128)
    b2p = jnp.tile(b2, 8).reshape(1, 128)
    blr = bl.reshape(1, 1)

    xpad = jnp.concatenate([x, jnp.zeros((NP - N, 17), _f32)])
    xp = xpad.reshape(NPK, XW)

    pad = EP - E
    src = jnp.concatenate([edge_index[0], jnp.zeros((pad,), jnp.int32)])
    dst = jnp.concatenate([edge_index[1], jnp.full((pad,), N, jnp.int32)])
    src2d = src.reshape(EP // BB, BB)
    dst2d = dst.reshape(EP // BB, BB)

    d0, d1 = _sc_deg(dst2d)
    g1p, dinvp = _tc_emb(xp, d0.reshape(NPK, 128), d1.reshape(NPK, 128), kmat, kqmat, c1p)
    a0, a1 = _sc_layer(g1p.reshape(NP, 16), src2d, dst2d)
    g2p = _tc_mid(g1p, a0.reshape(NPK, 128), a1.reshape(NPK, 128), dinvp, bd2, b1p)
    c0, c1x = _sc_layer(g2p.reshape(NP, 16), src2d, dst2d)
    y8 = _tc_head(g2p, c0.reshape(NPK, 128), c1x.reshape(NPK, 128), dinvp, smat, b2p, blr)
    return y8.reshape(NP, 1)[:N]
